# baseline (speedup 1.0000x reference)
import functools

import jax
import jax.numpy as jnp
from jax import lax
from jax.experimental import pallas as pl
from jax.experimental.pallas import tpu as pltpu
from jax.experimental.pallas import tpu_sc as plsc

N = 10000
IN_DIM = 256
OUT_DIM = 512
E = 160000

NP = 10240
EP = 163840
HALF = IN_DIM // 2

NC = 2
NS = 16
K = 64
NB = 4
STRIPE = NP // NS

_MESH = plsc.VectorSubcoreMesh(core_axis_name="c", subcore_axis_name="s")


def _fill_f32(ref, n, value):
    def body(i, _):
        ref[pl.ds(i * 16, 16)] = jnp.full((16,), value, jnp.float32)
        return 0
    lax.fori_loop(0, n // 16, body, 0)


CHUNKS = EP // (NS * K)
PHASES = 4
CPP = CHUNKS // PHASES
DEG_CHUNKS = EP // (NC * NS * K)


@functools.partial(
    pl.kernel,
    out_type=jax.ShapeDtypeStruct((NC, NP), jnp.float32),
    mesh=_MESH,
    scratch_types=[
        pltpu.VMEM((DEG_CHUNKS, K), jnp.int32),
        pltpu.VMEM((K,), jnp.float32),
        pltpu.VMEM((STRIPE,), jnp.float32),
        pltpu.VMEM_SHARED((NP,), jnp.float32),
    ],
)
def _deg_kernel(dst_hbm, deg_hbm, dst_all, ones_v, zbuf, deg_sh):
    c = lax.axis_index("c")
    s = lax.axis_index("s")
    _fill_f32(ones_v, K, 1.0)
    _fill_f32(zbuf, STRIPE, 0.0)
    pltpu.sync_copy(zbuf, deg_sh.at[pl.ds(s * STRIPE, STRIPE)])
    base_chunk = c * (NS * DEG_CHUNKS) + s * DEG_CHUNKS
    pltpu.sync_copy(dst_hbm.at[pl.ds(base_chunk, DEG_CHUNKS), :], dst_all)
    plsc.subcore_barrier()

    def body(j, _):
        pltpu.sync_copy(ones_v, deg_sh.at[dst_all.at[j]], add=True)
        return 0
    lax.fori_loop(0, DEG_CHUNKS, body, 0)

    plsc.subcore_barrier()
    pltpu.sync_copy(deg_sh.at[pl.ds(s * STRIPE, STRIPE)],
                    deg_hbm.at[c, pl.ds(s * STRIPE, STRIPE)])


R2 = 512


def _scale_body(x_ref, dega_ref, degb_ref, xs0_ref, xs1_ref, dinv_ref):
    i = pl.program_id(0)
    deg = dega_ref[...] + degb_ref[...] + 1.0
    dinv = lax.rsqrt(deg)
    row = i * R2 + lax.broadcasted_iota(jnp.int32, (R2, 1), 0)
    valid = row < N
    dinv = jnp.where(valid, dinv, 0.0)
    xs = jnp.where(valid, x_ref[...] * dinv, 0.0)
    xs0_ref[...] = xs[:, :HALF]
    xs1_ref[...] = xs[:, HALF:]
    dinv_ref[...] = dinv


_scale_kernel = pl.pallas_call(
    _scale_body,
    grid=(NP // R2,),
    in_specs=[
        pl.BlockSpec((R2, IN_DIM), lambda i: (i, 0)),
        pl.BlockSpec((R2, 1), lambda i: (i, 0)),
        pl.BlockSpec((R2, 1), lambda i: (i, 0)),
    ],
    out_specs=[
        pl.BlockSpec((R2, HALF), lambda i: (i, 0)),
        pl.BlockSpec((R2, HALF), lambda i: (i, 0)),
        pl.BlockSpec((R2, 1), lambda i: (i, 0)),
    ],
    out_shape=[
        jax.ShapeDtypeStruct((NP, HALF), jnp.float32),
        jax.ShapeDtypeStruct((NP, HALF), jnp.float32),
        jax.ShapeDtypeStruct((NP, 1), jnp.float32),
    ],
)


@functools.partial(
    pl.kernel,
    out_type=jax.ShapeDtypeStruct((NC, NP, HALF), jnp.float32),
    mesh=_MESH,
    scratch_types=[
        pltpu.VMEM((CPP, K), jnp.int32),
        pltpu.VMEM((CPP, K), jnp.int32),
        [pltpu.VMEM((K, HALF), jnp.float32)] * NB,
        [pltpu.SemaphoreType.DMA] * NB,
        [pltpu.SemaphoreType.DMA] * NB,
        pltpu.VMEM_SHARED((NP, HALF), jnp.float32),
    ],
)
def _agg_kernel(xs0_hbm, xs1_hbm, src_hbm, dst_hbm, s_out_hbm,
                src_all, dst_all, bufs, gsems, ssems, s_sh):
    c = lax.axis_index("c")
    s = lax.axis_index("s")

    def zfill(t, _):
        r = t // (HALF // 16)
        col = (t % (HALF // 16)) * 16
        bufs[0][r, pl.ds(col, 16)] = jnp.zeros((16,), jnp.float32)
        return 0
    lax.fori_loop(0, K * HALF // 16, zfill, 0)
    for k in range(STRIPE // K):
        pltpu.sync_copy(bufs[0], s_sh.at[pl.ds(s * STRIPE + k * K, K)])
    plsc.subcore_barrier()

    def run(xs_hbm, cid):
        def gather(j, b):
            pltpu.async_copy(xs_hbm.at[src_all.at[j]], bufs[b], gsems[b])

        def wait_gather(j, b):
            pltpu.make_async_copy(xs_hbm.at[src_all.at[j]], bufs[b],
                                  gsems[b]).wait()

        def scatter(j, b):
            pltpu.async_copy(bufs[b], s_sh.at[dst_all.at[j]], ssems[b],
                             add=True)

        def wait_scatter(j, b):
            pltpu.make_async_copy(bufs[b], s_sh.at[dst_all.at[j]],
                                  ssems[b]).wait()

        for p in range(PHASES):
            pltpu.sync_copy(src_hbm.at[pl.ds(s * CHUNKS + p * CPP, CPP), :],
                            src_all)
            pltpu.sync_copy(dst_hbm.at[pl.ds(s * CHUNKS + p * CPP, CPP), :],
                            dst_all)
            gather(0, 0)
            gather(1, 1)

            def body(i, _):
                for b in range(NB):
                    j = i * NB + b
                    wait_gather(j, b)
                    scatter(j, b)
                    bn = (b + 2) % NB

                    @pl.when(j + 2 < CPP)
                    def _():
                        @pl.when(j >= 2)
                        def _():
                            wait_scatter(j - 2, bn)
                        gather(j + 2, bn)
                return 0
            lax.fori_loop(0, CPP // NB, body, 0)
            for b in range(NB):
                wait_scatter(CPP - NB + b, (CPP - NB + b) % NB)

        plsc.subcore_barrier()
        pltpu.sync_copy(s_sh.at[pl.ds(s * STRIPE, STRIPE)],
                        s_out_hbm.at[cid, pl.ds(s * STRIPE, STRIPE), :])

    @pl.when(c == 0)
    def _():
        run(xs0_hbm, 0)

    @pl.when(c == 1)
    def _():
        run(xs1_hbm, 1)


R4 = 512


def _out_body(s_ref, xs0_ref, xs1_ref, dinv_ref, w_ref, b_ref, out_ref):
    dinv = dinv_ref[...]
    a0 = (s_ref[0] + xs0_ref[...]) * dinv
    a1 = (s_ref[1] + xs1_ref[...]) * dinv
    a = jnp.concatenate([a0, a1], axis=1)
    h = jnp.dot(a, w_ref[...], preferred_element_type=jnp.float32)
    h = jnp.maximum(h + b_ref[...], 0.0)
    m = jnp.max(h, axis=1, keepdims=True)
    e = jnp.exp(h - m)
    out_ref[...] = e / jnp.sum(e, axis=1, keepdims=True)


_out_kernel = pl.pallas_call(
    _out_body,
    grid=(NP // R4,),
    in_specs=[
        pl.BlockSpec((NC, R4, HALF), lambda i: (0, i, 0)),
        pl.BlockSpec((R4, HALF), lambda i: (i, 0)),
        pl.BlockSpec((R4, HALF), lambda i: (i, 0)),
        pl.BlockSpec((R4, 1), lambda i: (i, 0)),
        pl.BlockSpec((IN_DIM, OUT_DIM), lambda i: (0, 0)),
        pl.BlockSpec((1, OUT_DIM), lambda i: (0, 0)),
    ],
    out_specs=pl.BlockSpec((R4, OUT_DIM), lambda i: (i, 0)),
    out_shape=jax.ShapeDtypeStruct((N, OUT_DIM), jnp.float32),
)


def kernel(x, edge_index, W, b):
    src = edge_index[0].astype(jnp.int32)
    dst = edge_index[1].astype(jnp.int32)
    pad = jnp.full((EP - E,), N, dtype=jnp.int32)
    src_p = jnp.concatenate([src, pad]).reshape(EP // K, K)
    dst_p = jnp.concatenate([dst, pad]).reshape(EP // K, K)

    deg2 = _deg_kernel(dst_p)
    dega = deg2[0].reshape(NP, 1)
    degb = deg2[1].reshape(NP, 1)
    xs0, xs1, dinv = _scale_kernel(x, dega, degb)
    s_agg = _agg_kernel(xs0, xs1, src_p, dst_p)
    return _out_kernel(s_agg, xs0, xs1, dinv, W, b.reshape(1, OUT_DIM))


# speedup vs baseline: 1.0816x; 1.0816x over previous
import functools

import jax
import jax.numpy as jnp
from jax import lax
from jax.experimental import pallas as pl
from jax.experimental.pallas import tpu as pltpu
from jax.experimental.pallas import tpu_sc as plsc

N = 10000
IN_DIM = 256
OUT_DIM = 512
E = 160000

NP = 10240
EP = 163840
HALF = IN_DIM // 2

NC = 2
NS = 16
K = 80
NB = 4
STRIPE = NP // NS

_MESH = plsc.VectorSubcoreMesh(core_axis_name="c", subcore_axis_name="s")


def _fill_f32(ref, n, value):
    def body(i, _):
        ref[pl.ds(i * 16, 16)] = jnp.full((16,), value, jnp.float32)
        return 0
    lax.fori_loop(0, n // 16, body, 0)


CHUNKS = EP // (NS * K)
PHASES = 4
CPP = CHUNKS // PHASES
DEG_CHUNKS = EP // (NC * NS * K)


@functools.partial(
    pl.kernel,
    out_type=jax.ShapeDtypeStruct((NC, NP), jnp.float32),
    mesh=_MESH,
    scratch_types=[
        pltpu.VMEM((DEG_CHUNKS, K), jnp.int32),
        pltpu.VMEM((K,), jnp.float32),
        pltpu.VMEM((STRIPE,), jnp.float32),
        pltpu.VMEM_SHARED((NP,), jnp.float32),
    ],
)
def _deg_kernel(dst_hbm, deg_hbm, dst_all, ones_v, zbuf, deg_sh):
    c = lax.axis_index("c")
    s = lax.axis_index("s")
    _fill_f32(ones_v, K, 1.0)
    _fill_f32(zbuf, STRIPE, 0.0)
    pltpu.sync_copy(zbuf, deg_sh.at[pl.ds(s * STRIPE, STRIPE)])
    base_chunk = c * (NS * DEG_CHUNKS) + s * DEG_CHUNKS
    pltpu.sync_copy(dst_hbm.at[pl.ds(base_chunk, DEG_CHUNKS), :], dst_all)
    plsc.subcore_barrier()

    def body(j, _):
        pltpu.sync_copy(ones_v, deg_sh.at[dst_all.at[j]], add=True)
        return 0
    lax.fori_loop(0, DEG_CHUNKS, body, 0)

    plsc.subcore_barrier()
    pltpu.sync_copy(deg_sh.at[pl.ds(s * STRIPE, STRIPE)],
                    deg_hbm.at[c, pl.ds(s * STRIPE, STRIPE)])


R2 = 512


def _scale_body(x_ref, dega_ref, degb_ref, xs0_ref, xs1_ref, dinv_ref):
    i = pl.program_id(0)
    deg = dega_ref[...] + degb_ref[...] + 1.0
    dinv = lax.rsqrt(deg)
    row = i * R2 + lax.broadcasted_iota(jnp.int32, (R2, 1), 0)
    valid = row < N
    dinv = jnp.where(valid, dinv, 0.0)
    xs = jnp.where(valid, x_ref[...] * dinv, 0.0)
    xs0_ref[...] = xs[:, :HALF]
    xs1_ref[...] = xs[:, HALF:]
    dinv_ref[...] = dinv


_scale_kernel = pl.pallas_call(
    _scale_body,
    grid=(NP // R2,),
    in_specs=[
        pl.BlockSpec((R2, IN_DIM), lambda i: (i, 0)),
        pl.BlockSpec((R2, 1), lambda i: (i, 0)),
        pl.BlockSpec((R2, 1), lambda i: (i, 0)),
    ],
    out_specs=[
        pl.BlockSpec((R2, HALF), lambda i: (i, 0)),
        pl.BlockSpec((R2, HALF), lambda i: (i, 0)),
        pl.BlockSpec((R2, 1), lambda i: (i, 0)),
    ],
    out_shape=[
        jax.ShapeDtypeStruct((NP, HALF), jnp.float32),
        jax.ShapeDtypeStruct((NP, HALF), jnp.float32),
        jax.ShapeDtypeStruct((NP, 1), jnp.float32),
    ],
)


@functools.partial(
    pl.kernel,
    out_type=jax.ShapeDtypeStruct((NC, NP, HALF), jnp.float32),
    mesh=_MESH,
    scratch_types=[
        pltpu.VMEM((CPP, K), jnp.int32),
        pltpu.VMEM((CPP, K), jnp.int32),
        [pltpu.VMEM((K, HALF), jnp.float32)] * NB,
        [pltpu.SemaphoreType.DMA] * NB,
        [pltpu.SemaphoreType.DMA] * NB,
        pltpu.VMEM_SHARED((NP, HALF), jnp.float32),
    ],
)
def _agg_kernel(xs0_hbm, xs1_hbm, src_hbm, dst_hbm, s_out_hbm,
                src_all, dst_all, bufs, gsems, ssems, s_sh):
    c = lax.axis_index("c")
    s = lax.axis_index("s")

    def zfill(t, _):
        r = t // (HALF // 16)
        col = (t % (HALF // 16)) * 16
        bufs[0][r, pl.ds(col, 16)] = jnp.zeros((16,), jnp.float32)
        return 0
    lax.fori_loop(0, K * HALF // 16, zfill, 0)
    for k in range(STRIPE // K):
        pltpu.sync_copy(bufs[0], s_sh.at[pl.ds(s * STRIPE + k * K, K)])
    plsc.subcore_barrier()

    def run(xs_hbm, cid):
        def gather(j, b):
            pltpu.async_copy(xs_hbm.at[src_all.at[j]], bufs[b], gsems[b])

        def wait_gather(j, b):
            pltpu.make_async_copy(xs_hbm.at[src_all.at[j]], bufs[b],
                                  gsems[b]).wait()

        def scatter(j, b):
            pltpu.async_copy(bufs[b], s_sh.at[dst_all.at[j]], ssems[b],
                             add=True)

        def wait_scatter(j, b):
            pltpu.make_async_copy(bufs[b], s_sh.at[dst_all.at[j]],
                                  ssems[b]).wait()

        for p in range(PHASES):
            pltpu.sync_copy(src_hbm.at[pl.ds(s * CHUNKS + p * CPP, CPP), :],
                            src_all)
            pltpu.sync_copy(dst_hbm.at[pl.ds(s * CHUNKS + p * CPP, CPP), :],
                            dst_all)
            gather(0, 0)
            gather(1, 1)

            def body(i, _):
                for b in range(NB):
                    j = i * NB + b
                    wait_gather(j, b)
                    scatter(j, b)
                    bn = (b + 2) % NB

                    @pl.when(j + 2 < CPP)
                    def _():
                        @pl.when(j >= 2)
                        def _():
                            wait_scatter(j - 2, bn)
                        gather(j + 2, bn)
                return 0
            lax.fori_loop(0, CPP // NB, body, 0)
            for b in range(NB):
                wait_scatter(CPP - NB + b, (CPP - NB + b) % NB)

        plsc.subcore_barrier()
        pltpu.sync_copy(s_sh.at[pl.ds(s * STRIPE, STRIPE)],
                        s_out_hbm.at[cid, pl.ds(s * STRIPE, STRIPE), :])

    @pl.when(c == 0)
    def _():
        run(xs0_hbm, 0)

    @pl.when(c == 1)
    def _():
        run(xs1_hbm, 1)


R4 = 512


def _out_body(s_ref, xs0_ref, xs1_ref, dinv_ref, w_ref, b_ref, out_ref):
    dinv = dinv_ref[...]
    a0 = (s_ref[0] + xs0_ref[...]) * dinv
    a1 = (s_ref[1] + xs1_ref[...]) * dinv
    a = jnp.concatenate([a0, a1], axis=1)
    h = jnp.dot(a, w_ref[...], preferred_element_type=jnp.float32)
    h = jnp.maximum(h + b_ref[...], 0.0)
    m = jnp.max(h, axis=1, keepdims=True)
    e = jnp.exp(h - m)
    out_ref[...] = e / jnp.sum(e, axis=1, keepdims=True)


_out_kernel = pl.pallas_call(
    _out_body,
    grid=(NP // R4,),
    in_specs=[
        pl.BlockSpec((NC, R4, HALF), lambda i: (0, i, 0)),
        pl.BlockSpec((R4, HALF), lambda i: (i, 0)),
        pl.BlockSpec((R4, HALF), lambda i: (i, 0)),
        pl.BlockSpec((R4, 1), lambda i: (i, 0)),
        pl.BlockSpec((IN_DIM, OUT_DIM), lambda i: (0, 0)),
        pl.BlockSpec((1, OUT_DIM), lambda i: (0, 0)),
    ],
    out_specs=pl.BlockSpec((R4, OUT_DIM), lambda i: (i, 0)),
    out_shape=jax.ShapeDtypeStruct((N, OUT_DIM), jnp.float32),
)


def kernel(x, edge_index, W, b):
    src = edge_index[0].astype(jnp.int32)
    dst = edge_index[1].astype(jnp.int32)
    pad = jnp.full((EP - E,), N, dtype=jnp.int32)
    src_p = jnp.concatenate([src, pad]).reshape(EP // K, K)
    dst_p = jnp.concatenate([dst, pad]).reshape(EP // K, K)

    deg2 = _deg_kernel(dst_p)
    dega = deg2[0].reshape(NP, 1)
    degb = deg2[1].reshape(NP, 1)
    xs0, xs1, dinv = _scale_kernel(x, dega, degb)
    s_agg = _agg_kernel(xs0, xs1, src_p, dst_p)
    return _out_kernel(s_agg, xs0, xs1, dinv, W, b.reshape(1, OUT_DIM))


# speedup vs baseline: 1.2138x; 1.1222x over previous
import functools

import jax
import jax.numpy as jnp
from jax import lax
from jax.experimental import pallas as pl
from jax.experimental.pallas import tpu as pltpu
from jax.experimental.pallas import tpu_sc as plsc

N = 10000
IN_DIM = 256
OUT_DIM = 512
E = 160000

NP = 10240
EP = 163840
HALF = IN_DIM // 2

NC = 2
NS = 16
K = 128
NB = 2
STRIPE = NP // NS

_MESH = plsc.VectorSubcoreMesh(core_axis_name="c", subcore_axis_name="s")


def _fill_f32(ref, n, value):
    def body(i, _):
        ref[pl.ds(i * 16, 16)] = jnp.full((16,), value, jnp.float32)
        return 0
    lax.fori_loop(0, n // 16, body, 0)


CHUNKS = EP // (NS * K)
PHASES = 2
CPP = CHUNKS // PHASES
DEG_CHUNKS = EP // (NC * NS * K)


@functools.partial(
    pl.kernel,
    out_type=jax.ShapeDtypeStruct((NC, NP), jnp.float32),
    mesh=_MESH,
    scratch_types=[
        pltpu.VMEM((DEG_CHUNKS, K), jnp.int32),
        pltpu.VMEM((K,), jnp.float32),
        pltpu.VMEM((STRIPE,), jnp.float32),
        pltpu.VMEM_SHARED((NP,), jnp.float32),
    ],
)
def _deg_kernel(dst_hbm, deg_hbm, dst_all, ones_v, zbuf, deg_sh):
    c = lax.axis_index("c")
    s = lax.axis_index("s")
    _fill_f32(ones_v, K, 1.0)
    _fill_f32(zbuf, STRIPE, 0.0)
    pltpu.sync_copy(zbuf, deg_sh.at[pl.ds(s * STRIPE, STRIPE)])
    base_chunk = c * (NS * DEG_CHUNKS) + s * DEG_CHUNKS
    pltpu.sync_copy(dst_hbm.at[pl.ds(base_chunk, DEG_CHUNKS), :], dst_all)
    plsc.subcore_barrier()

    def body(j, _):
        pltpu.sync_copy(ones_v, deg_sh.at[dst_all.at[j]], add=True)
        return 0
    lax.fori_loop(0, DEG_CHUNKS, body, 0)

    plsc.subcore_barrier()
    pltpu.sync_copy(deg_sh.at[pl.ds(s * STRIPE, STRIPE)],
                    deg_hbm.at[c, pl.ds(s * STRIPE, STRIPE)])


R2 = 512


def _scale_body(x_ref, dega_ref, degb_ref, xs0_ref, xs1_ref, dinv_ref):
    i = pl.program_id(0)
    deg = dega_ref[...] + degb_ref[...] + 1.0
    dinv = lax.rsqrt(deg)
    row = i * R2 + lax.broadcasted_iota(jnp.int32, (R2, 1), 0)
    valid = row < N
    dinv = jnp.where(valid, dinv, 0.0)
    xs = jnp.where(valid, x_ref[...] * dinv, 0.0)
    xs0_ref[...] = xs[:, :HALF]
    xs1_ref[...] = xs[:, HALF:]
    dinv_ref[...] = dinv


_scale_kernel = pl.pallas_call(
    _scale_body,
    grid=(NP // R2,),
    in_specs=[
        pl.BlockSpec((R2, IN_DIM), lambda i: (i, 0)),
        pl.BlockSpec((R2, 1), lambda i: (i, 0)),
        pl.BlockSpec((R2, 1), lambda i: (i, 0)),
    ],
    out_specs=[
        pl.BlockSpec((R2, HALF), lambda i: (i, 0)),
        pl.BlockSpec((R2, HALF), lambda i: (i, 0)),
        pl.BlockSpec((R2, 1), lambda i: (i, 0)),
    ],
    out_shape=[
        jax.ShapeDtypeStruct((NP, HALF), jnp.float32),
        jax.ShapeDtypeStruct((NP, HALF), jnp.float32),
        jax.ShapeDtypeStruct((NP, 1), jnp.float32),
    ],
)


@functools.partial(
    pl.kernel,
    out_type=jax.ShapeDtypeStruct((NC, NP, HALF), jnp.float32),
    mesh=_MESH,
    scratch_types=[
        pltpu.VMEM((CPP, K), jnp.int32),
        pltpu.VMEM((CPP, K), jnp.int32),
        [pltpu.VMEM((K, HALF), jnp.float32)] * NB,
        [pltpu.SemaphoreType.DMA] * NB,
        pltpu.VMEM_SHARED((NP, HALF), jnp.float32),
    ],
)
def _agg_kernel(xs0_hbm, xs1_hbm, src_hbm, dst_hbm, s_out_hbm,
                src_all, dst_all, bufs, gsems, s_sh):
    c = lax.axis_index("c")
    s = lax.axis_index("s")

    def zfill(t, _):
        r = t // (HALF // 16)
        col = (t % (HALF // 16)) * 16
        bufs[0][r, pl.ds(col, 16)] = jnp.zeros((16,), jnp.float32)
        return 0
    lax.fori_loop(0, K * HALF // 16, zfill, 0)
    for k in range(STRIPE // K):
        pltpu.sync_copy(bufs[0], s_sh.at[pl.ds(s * STRIPE + k * K, K)])
    plsc.subcore_barrier()

    def run(xs_hbm, cid):
        def gather(j, b):
            pltpu.async_copy(xs_hbm.at[src_all.at[j]], bufs[b], gsems[b])

        def wait_gather(j, b):
            pltpu.make_async_copy(xs_hbm.at[src_all.at[j]], bufs[b],
                                  gsems[b]).wait()

        for p in range(PHASES):
            pltpu.sync_copy(src_hbm.at[pl.ds(s * CHUNKS + p * CPP, CPP), :],
                            src_all)
            pltpu.sync_copy(dst_hbm.at[pl.ds(s * CHUNKS + p * CPP, CPP), :],
                            dst_all)
            gather(0, 0)
            gather(1, 1)

            def body(i, _):
                for b in range(NB):
                    j = i * NB + b
                    wait_gather(j, b)
                    pltpu.sync_copy(bufs[b], s_sh.at[dst_all.at[j]], add=True)

                    @pl.when(j + 2 < CPP)
                    def _():
                        gather(j + 2, b)
                return 0
            lax.fori_loop(0, CPP // NB, body, 0)

        plsc.subcore_barrier()
        pltpu.sync_copy(s_sh.at[pl.ds(s * STRIPE, STRIPE)],
                        s_out_hbm.at[cid, pl.ds(s * STRIPE, STRIPE), :])

    @pl.when(c == 0)
    def _():
        run(xs0_hbm, 0)

    @pl.when(c == 1)
    def _():
        run(xs1_hbm, 1)


R4 = 512


def _out_body(s_ref, x_ref, dinv_ref, w_ref, b_ref, out_ref):
    dinv = dinv_ref[...]
    s01 = jnp.concatenate([s_ref[0], s_ref[1]], axis=1).astype(jnp.float32)
    a = s01 * dinv + x_ref[...] * (dinv * dinv)
    h = jnp.dot(a, w_ref[...], preferred_element_type=jnp.float32)
    h = jnp.maximum(h + b_ref[...], 0.0)
    m = jnp.max(h, axis=1, keepdims=True)
    e = jnp.exp(h - m)
    out_ref[...] = e / jnp.sum(e, axis=1, keepdims=True)


_out_kernel = pl.pallas_call(
    _out_body,
    grid=(NP // R4,),
    in_specs=[
        pl.BlockSpec((NC, R4, HALF), lambda i: (0, i, 0)),
        pl.BlockSpec((R4, IN_DIM), lambda i: (i, 0)),
        pl.BlockSpec((R4, 1), lambda i: (i, 0)),
        pl.BlockSpec((IN_DIM, OUT_DIM), lambda i: (0, 0)),
        pl.BlockSpec((1, OUT_DIM), lambda i: (0, 0)),
    ],
    out_specs=pl.BlockSpec((R4, OUT_DIM), lambda i: (i, 0)),
    out_shape=jax.ShapeDtypeStruct((N, OUT_DIM), jnp.float32),
)


def kernel(x, edge_index, W, b):
    src = edge_index[0].astype(jnp.int32)
    dst = edge_index[1].astype(jnp.int32)
    pad = jnp.full((EP - E,), N, dtype=jnp.int32)
    src_p = jnp.concatenate([src, pad]).reshape(EP // K, K)
    dst_p = jnp.concatenate([dst, pad]).reshape(EP // K, K)

    deg2 = _deg_kernel(dst_p)
    dega = deg2[0].reshape(NP, 1)
    degb = deg2[1].reshape(NP, 1)
    xs0, xs1, dinv = _scale_kernel(x, dega, degb)
    s_agg = _agg_kernel(xs0, xs1, src_p, dst_p)
    return _out_kernel(s_agg, x, dinv, W, b.reshape(1, OUT_DIM))
